# Initial kernel scaffold; baseline (speedup 1.0000x reference)
import jax
import jax.numpy as jnp
from jax.experimental import pallas as pl


def kernel(x, edge_index, batch, y, perm, W_gcn, b_gcn, W_ctx, b_ctx, W_obj, b_obj, W_ib, b_ib, W1, b1, W4, b4, W3, b3, Wc1, bc1, Wc2, bc2, Wo1, bo1, Wo2, bo2, Ws1, bs1, Ws2, bs2):
    raise NotImplementedError("write your pallas kernel here")


# speedup vs baseline: 12.0200x; 12.0200x over previous
import functools

import jax
import jax.numpy as jnp
from jax import lax
from jax.experimental import pallas as pl
from jax.experimental.pallas import tpu as pltpu
from jax.experimental.pallas import tpu_sc as plsc

N, E, D, H, G = 10000, 320000, 128, 64, 128
NC, NS = 2, 16
NW = NC * NS
K = 80

_f32 = jnp.float32


def _sc_mesh():
    return plsc.VectorSubcoreMesh(
        core_axis_name="c", subcore_axis_name="s", num_cores=NC, num_subcores=NS
    )


def _edge_sweep(table_hbm, src_hbm, dst_hbm, acc, sidx, didx, rows, sem,
                ebase, nblk):
    def body(i, carry):
        base = ebase + i * K
        pltpu.sync_copy(src_hbm.at[pl.ds(base, K)], sidx)
        pltpu.sync_copy(dst_hbm.at[pl.ds(base, K)], didx)
        pltpu.async_copy(table_hbm.at[sidx], rows, sem).wait()
        pltpu.sync_copy(rows, acc.at[didx], add=True)
        return carry
    lax.fori_loop(0, nblk, body, 0, unroll=False)


_WCH = 624
_WLAST = N - (NS - 1) * _WCH


def _writeback(acc, out_hbm, c, s):
    @pl.when(s < NS - 1)
    def _():
        pltpu.sync_copy(acc.at[pl.ds(s * _WCH, _WCH)],
                        out_hbm.at[c, pl.ds(s * _WCH, _WCH)])

    @pl.when(s == NS - 1)
    def _():
        pltpu.sync_copy(acc.at[pl.ds((NS - 1) * _WCH, _WLAST)],
                        out_hbm.at[c, pl.ds((NS - 1) * _WCH, _WLAST)])


def _make_segsum1():
    epw = E // NW
    nblk = epw // K

    @functools.partial(
        pl.kernel,
        out_type=jax.ShapeDtypeStruct((NC, N, D), _f32),
        mesh=_sc_mesh(),
        scratch_types=[
            pltpu.VMEM((K,), jnp.int32),
            pltpu.VMEM((K,), jnp.int32),
            pltpu.VMEM((K, D), _f32),
            pltpu.VMEM_SHARED((N, D), _f32),
            pltpu.SemaphoreType.DMA,
        ],
    )
    def segsum1(x_hbm, src_hbm, dst_hbm, zeros_hbm, out_hbm,
                sidx, didx, rows, acc, sem):
        c = lax.axis_index("c")
        s = lax.axis_index("s")

        @pl.when(s == 0)
        def _():
            pltpu.sync_copy(zeros_hbm, acc)
        plsc.subcore_barrier()

        wid = c * NS + s
        _edge_sweep(x_hbm, src_hbm, dst_hbm, acc, sidx, didx, rows, sem,
                    wid * epw, nblk)
        plsc.subcore_barrier()
        _writeback(acc, out_hbm, c, s)

    return segsum1


def _make_segsum2():
    ept = E // NS
    nblk = ept // K

    @functools.partial(
        pl.kernel,
        out_type=jax.ShapeDtypeStruct((NC, N, D), _f32),
        mesh=_sc_mesh(),
        scratch_types=[
            pltpu.VMEM((K,), jnp.int32),
            pltpu.VMEM((K,), jnp.int32),
            pltpu.VMEM((K, D), _f32),
            pltpu.VMEM_SHARED((N, D), _f32),
            pltpu.SemaphoreType.DMA,
        ],
    )
    def segsum2(xs0_hbm, xs1_hbm, src_hbm, dst_hbm, zeros_hbm, out_hbm,
                sidx, didx, rows, acc, sem):
        c = lax.axis_index("c")
        s = lax.axis_index("s")

        @pl.when(s == 0)
        def _():
            pltpu.sync_copy(zeros_hbm, acc)
        plsc.subcore_barrier()

        ebase = s * ept

        @pl.when(c == 0)
        def _():
            _edge_sweep(xs0_hbm, src_hbm, dst_hbm, acc, sidx, didx, rows, sem,
                        ebase, nblk)

        @pl.when(c == 1)
        def _():
            _edge_sweep(xs1_hbm, src_hbm, dst_hbm, acc, sidx, didx, rows, sem,
                        ebase, nblk)
        plsc.subcore_barrier()
        _writeback(acc, out_hbm, c, s)

    return segsum2


_R = 1000
_NB = N // _R


def _onehot_batch(batch_ref, r):
    b = batch_ref[0, 0, :].astype(_f32)
    diff = b[:, None] - lax.broadcasted_iota(jnp.int32, (r, G), 1).astype(_f32)
    return jnp.maximum(1.0 - jnp.abs(diff), 0.0)


def _tc1_body(a1a, a1b, x_ref, batch_ref, wg, bg, wib, bib,
              asn_ref, xs0_ref, xs1_ref, g1_ref):
    i = pl.program_id(0)
    agg1 = a1a[...] + a1b[...]
    z1 = jnp.maximum(agg1 @ wg[...] + bg[...], 0.0)
    logit = z1 @ wib[...] + bib[...]
    m = jnp.max(logit, axis=1, keepdims=True)
    ex = jnp.exp(logit - m)
    a = ex / jnp.sum(ex, axis=1, keepdims=True)
    asn_ref[...] = a
    xv = x_ref[...]
    xs0_ref[...] = xv * a[:, 0:1]
    xs1_ref[...] = xv * a[:, 1:2]
    oh = _onehot_batch(batch_ref, _R)
    contrib = lax.dot_general(oh, z1, (((0,), (0,)), ((), ())))

    @pl.when(i == 0)
    def _():
        g1_ref[...] = jnp.zeros_like(g1_ref)
    g1_ref[...] += contrib


def _tc2_body(s0_ref, s1_ref, asn_ref, batch_ref, wc, bc, wo, bo,
              gm_ref, gres_ref):
    i = pl.program_id(0)
    a = asn_ref[...]
    agg_res = s0_ref[...] * a[:, 0:1]
    agg_m = s1_ref[...] * a[:, 1:2]
    z_m = jnp.maximum(agg_m @ wc[...] + bc[...], 0.0)
    z_res = jnp.maximum(agg_res @ wo[...] + bo[...], 0.0)
    oh = _onehot_batch(batch_ref, _R)

    @pl.when(i == 0)
    def _():
        gm_ref[...] = jnp.zeros_like(gm_ref)
        gres_ref[...] = jnp.zeros_like(gres_ref)
    gm_ref[...] += lax.dot_general(oh, z_m, (((0,), (0,)), ((), ())))
    gres_ref[...] += lax.dot_general(oh, z_res, (((0,), (0,)), ((), ())))


def _tc3_body(g1, gm, gres, pmat_ref,
              w1, b1, w4, b4, wc1, bc1, wc2, bc2,
              wo1, bo1, wo2, bo2, ws1, bs1, ws2, bs2,
              hco_ref, hm_ref, hshuf_ref, h1_ref, hres_ref):
    p = pmat_ref[...]
    g1v, gmv, gresv = g1[...], gm[...], gres[...]
    g_co = p @ gresv + gmv
    g_co_s = gresv + p @ gmv
    hco_ref[...] = jnp.maximum(g_co @ wc1[...] + bc1[...], 0.0) @ wc2[...] + bc2[...]
    hm_ref[...] = jnp.maximum(gmv @ wo1[...] + bo1[...], 0.0) @ wo2[...] + bo2[...]
    hshuf_ref[...] = jnp.maximum(g_co_s @ ws1[...] + bs1[...], 0.0) @ ws2[...] + bs2[...]
    h1_ref[...] = g1v @ w1[...] + b1[...]
    hres_ref[...] = gresv @ w4[...] + b4[...]


def _tc4_body(g1, gm, w3, b3, p1_ref, pm_ref):
    pm_ref[...] = gm[...] @ w3[...] + b3[...]
    p1_ref[...] = g1[...] @ w3[...] + b3[...]


def _row_spec(r, d):
    return pl.BlockSpec((r, d), lambda i: (i, 0))


def _full_spec(shape):
    nd = len(shape)
    return pl.BlockSpec(shape, lambda i: (0,) * nd)


def _tc1_call(a1a, a1b, x, batch3d, wg, bg, wib, bib):
    return pl.pallas_call(
        _tc1_body,
        grid=(_NB,),
        in_specs=[
            _row_spec(_R, D), _row_spec(_R, D), _row_spec(_R, D),
            pl.BlockSpec((1, 1, _R), lambda i: (i, 0, 0)),
            _full_spec((D, H)), _full_spec((1, H)),
            _full_spec((H, 2)), _full_spec((1, 2)),
        ],
        out_specs=[
            _row_spec(_R, 2), _row_spec(_R, D), _row_spec(_R, D),
            _full_spec((G, H)),
        ],
        out_shape=[
            jax.ShapeDtypeStruct((N, 2), _f32),
            jax.ShapeDtypeStruct((N, D), _f32),
            jax.ShapeDtypeStruct((N, D), _f32),
            jax.ShapeDtypeStruct((G, H), _f32),
        ],
    )(a1a, a1b, x, batch3d, wg, bg, wib, bib)


def _tc2_call(s0, s1, asn, batch3d, wc, bc, wo, bo):
    return pl.pallas_call(
        _tc2_body,
        grid=(_NB,),
        in_specs=[
            _row_spec(_R, D), _row_spec(_R, D), _row_spec(_R, 2),
            pl.BlockSpec((1, 1, _R), lambda i: (i, 0, 0)),
            _full_spec((D, H)), _full_spec((1, H)),
            _full_spec((D, H)), _full_spec((1, H)),
        ],
        out_specs=[_full_spec((G, H)), _full_spec((G, H))],
        out_shape=[
            jax.ShapeDtypeStruct((G, H), _f32),
            jax.ShapeDtypeStruct((G, H), _f32),
        ],
    )(s0, s1, asn, batch3d, wc, bc, wo, bo)


def _tc3_call(g1, gm, gres, pmat, *weights):
    two = 2
    h_co, h_m, h_shuf, h1, h_res = pl.pallas_call(
        _tc3_body,
        out_shape=[
            jax.ShapeDtypeStruct((G, two), _f32),
            jax.ShapeDtypeStruct((G, two), _f32),
            jax.ShapeDtypeStruct((G, two), _f32),
            jax.ShapeDtypeStruct((G, two), _f32),
            jax.ShapeDtypeStruct((G, two), _f32),
        ],
    )(g1, gm, gres, pmat, *weights)
    return h1, h_m, h_co, h_res, h_shuf


def _tc4_call(g1, gm, w3, b3):
    return pl.pallas_call(
        _tc4_body,
        out_shape=[
            jax.ShapeDtypeStruct((G, H), _f32),
            jax.ShapeDtypeStruct((G, H), _f32),
        ],
    )(g1, gm, w3, b3)


def kernel(x, edge_index, batch, y, perm,
           W_gcn, b_gcn, W_ctx, b_ctx, W_obj, b_obj,
           W_ib, b_ib, W1, b1, W4, b4, W3, b3,
           Wc1, bc1, Wc2, bc2, Wo1, bo1, Wo2, bo2, Ws1, bs1, Ws2, bs2):
    src = edge_index[0]
    dst = edge_index[1]
    zeros_nd = jnp.zeros((N, D), _f32)
    batch3d = batch.reshape(_NB, 1, _R)
    r2 = lambda b: b.reshape(1, -1)

    agg_parts = _make_segsum1()(x, src, dst, zeros_nd)
    asn, xs0, xs1, g1 = _tc1_call(
        agg_parts[0], agg_parts[1], x, batch3d,
        W_gcn, r2(b_gcn), W_ib, r2(b_ib))

    s_parts = _make_segsum2()(xs0, xs1, src, dst, zeros_nd)
    g_m, g_res = _tc2_call(
        s_parts[0], s_parts[1], asn, batch3d,
        W_ctx, r2(b_ctx), W_obj, r2(b_obj))

    diff = perm.astype(_f32)[:, None] - jnp.arange(G, dtype=_f32)[None, :]
    pmat = jnp.maximum(1.0 - jnp.abs(diff), 0.0)
    h1, h_m, h_co, h_res, h_shuf = _tc3_call(
        g1, g_m, g_res, pmat,
        W1, r2(b1), W4, r2(b4),
        Wc1, r2(bc1), Wc2, r2(bc2), Wo1, r2(bo1), Wo2, r2(bo2),
        Ws1, r2(bs1), Ws2, r2(bs2))
    proj1, proj_m = _tc4_call(g1, g_m, W3, r2(b3))

    y_shuf = jnp.take(y, perm, axis=0)
    return (h1, g1, h_m, g_m, proj1, proj_m, asn, h_co, h_res, h_shuf, y_shuf)


# speedup vs baseline: 16.8816x; 1.4045x over previous
import functools

import jax
import jax.numpy as jnp
from jax import lax
from jax.experimental import pallas as pl
from jax.experimental.pallas import tpu as pltpu
from jax.experimental.pallas import tpu_sc as plsc

N, E, D, H, G = 10000, 320000, 128, 64, 128
NC, NS = 2, 16
NW = NC * NS
K = 80

_f32 = jnp.float32


def _sc_mesh():
    return plsc.VectorSubcoreMesh(
        core_axis_name="c", subcore_axis_name="s", num_cores=NC, num_subcores=NS
    )


def _edge_sweep(table_hbm, src_hbm, dst_hbm, acc, sidx, didx, rows, sem,
                ebase, nblk):
    def body(i, carry):
        base = ebase + i * K
        pltpu.sync_copy(src_hbm.at[pl.ds(base, K)], sidx)
        pltpu.sync_copy(dst_hbm.at[pl.ds(base, K)], didx)
        pltpu.async_copy(table_hbm.at[sidx], rows, sem).wait()
        pltpu.sync_copy(rows, acc.at[didx], add=True)
        return carry
    lax.fori_loop(0, nblk, body, 0, unroll=False)


_WCH = 624
_WLAST = N - (NS - 1) * _WCH


def _writeback(acc, out_hbm, c, s):
    @pl.when(s < NS - 1)
    def _():
        pltpu.sync_copy(acc.at[pl.ds(s * _WCH, _WCH)],
                        out_hbm.at[c, pl.ds(s * _WCH, _WCH)])

    @pl.when(s == NS - 1)
    def _():
        pltpu.sync_copy(acc.at[pl.ds((NS - 1) * _WCH, _WLAST)],
                        out_hbm.at[c, pl.ds((NS - 1) * _WCH, _WLAST)])


def _make_segsum1():
    epw = E // NW
    nblk = epw // K

    @functools.partial(
        pl.kernel,
        out_type=jax.ShapeDtypeStruct((NC, N, D), _f32),
        mesh=_sc_mesh(),
        scratch_types=[
            pltpu.VMEM((K,), jnp.int32),
            pltpu.VMEM((K,), jnp.int32),
            pltpu.VMEM((K, D), _f32),
            pltpu.VMEM_SHARED((N, D), _f32),
            pltpu.SemaphoreType.DMA,
        ],
    )
    def segsum1(x_hbm, src_hbm, dst_hbm, zeros_hbm, out_hbm,
                sidx, didx, rows, acc, sem):
        c = lax.axis_index("c")
        s = lax.axis_index("s")

        @pl.when(s == 0)
        def _():
            pltpu.sync_copy(zeros_hbm, acc)
        plsc.subcore_barrier()

        wid = c * NS + s
        _edge_sweep(x_hbm, src_hbm, dst_hbm, acc, sidx, didx, rows, sem,
                    wid * epw, nblk)
        plsc.subcore_barrier()
        _writeback(acc, out_hbm, c, s)

    return segsum1


_R = 1000
_NB = N // _R


def _onehot_batch(batch_ref, r):
    b = batch_ref[0, 0, :].astype(_f32)
    diff = b[:, None] - lax.broadcasted_iota(jnp.int32, (r, G), 1).astype(_f32)
    return jnp.maximum(1.0 - jnp.abs(diff), 0.0)


def _tc1_body(a1a, a1b, x_ref, batch_ref, wg, bg, wib, bib,
              asn_ref, xs0_ref, agg1_ref, g1_ref):
    i = pl.program_id(0)
    agg1 = a1a[...] + a1b[...]
    agg1_ref[...] = agg1
    z1 = jnp.maximum(agg1 @ wg[...] + bg[...], 0.0)
    logit = z1 @ wib[...] + bib[...]
    m = jnp.max(logit, axis=1, keepdims=True)
    ex = jnp.exp(logit - m)
    a = ex / jnp.sum(ex, axis=1, keepdims=True)
    asn_ref[...] = a
    xs0_ref[...] = x_ref[...] * a[:, 0:1]
    oh = _onehot_batch(batch_ref, _R)
    contrib = lax.dot_general(oh, z1, (((0,), (0,)), ((), ())))

    @pl.when(i == 0)
    def _():
        g1_ref[...] = jnp.zeros_like(g1_ref)
    g1_ref[...] += contrib


def _tc2_body(s0a_ref, s0b_ref, agg1_ref, asn_ref, batch_ref, wc, bc, wo, bo,
              gm_ref, gres_ref):
    i = pl.program_id(0)
    a = asn_ref[...]
    s0 = s0a_ref[...] + s0b_ref[...]
    agg_res = s0 * a[:, 0:1]
    agg_m = (agg1_ref[...] - s0) * a[:, 1:2]
    z_m = jnp.maximum(agg_m @ wc[...] + bc[...], 0.0)
    z_res = jnp.maximum(agg_res @ wo[...] + bo[...], 0.0)
    oh = _onehot_batch(batch_ref, _R)

    @pl.when(i == 0)
    def _():
        gm_ref[...] = jnp.zeros_like(gm_ref)
        gres_ref[...] = jnp.zeros_like(gres_ref)
    gm_ref[...] += lax.dot_general(oh, z_m, (((0,), (0,)), ((), ())))
    gres_ref[...] += lax.dot_general(oh, z_res, (((0,), (0,)), ((), ())))


def _tc3_body(g1, gm, gres, pmat_ref,
              w1, b1, w4, b4, wc1, bc1, wc2, bc2,
              wo1, bo1, wo2, bo2, ws1, bs1, ws2, bs2,
              hco_ref, hm_ref, hshuf_ref, h1_ref, hres_ref):
    p = pmat_ref[...]
    g1v, gmv, gresv = g1[...], gm[...], gres[...]
    g_co = p @ gresv + gmv
    g_co_s = gresv + p @ gmv
    hco_ref[...] = jnp.maximum(g_co @ wc1[...] + bc1[...], 0.0) @ wc2[...] + bc2[...]
    hm_ref[...] = jnp.maximum(gmv @ wo1[...] + bo1[...], 0.0) @ wo2[...] + bo2[...]
    hshuf_ref[...] = jnp.maximum(g_co_s @ ws1[...] + bs1[...], 0.0) @ ws2[...] + bs2[...]
    h1_ref[...] = g1v @ w1[...] + b1[...]
    hres_ref[...] = gresv @ w4[...] + b4[...]


def _tc4_body(g1, gm, w3, b3, p1_ref, pm_ref):
    pm_ref[...] = gm[...] @ w3[...] + b3[...]
    p1_ref[...] = g1[...] @ w3[...] + b3[...]


def _row_spec(r, d):
    return pl.BlockSpec((r, d), lambda i: (i, 0))


def _full_spec(shape):
    nd = len(shape)
    return pl.BlockSpec(shape, lambda i: (0,) * nd)


def _tc1_call(a1a, a1b, x, batch3d, wg, bg, wib, bib):
    return pl.pallas_call(
        _tc1_body,
        grid=(_NB,),
        in_specs=[
            _row_spec(_R, D), _row_spec(_R, D), _row_spec(_R, D),
            pl.BlockSpec((1, 1, _R), lambda i: (i, 0, 0)),
            _full_spec((D, H)), _full_spec((1, H)),
            _full_spec((H, 2)), _full_spec((1, 2)),
        ],
        out_specs=[
            _row_spec(_R, 2), _row_spec(_R, D), _row_spec(_R, D),
            _full_spec((G, H)),
        ],
        out_shape=[
            jax.ShapeDtypeStruct((N, 2), _f32),
            jax.ShapeDtypeStruct((N, D), _f32),
            jax.ShapeDtypeStruct((N, D), _f32),
            jax.ShapeDtypeStruct((G, H), _f32),
        ],
    )(a1a, a1b, x, batch3d, wg, bg, wib, bib)


def _tc2_call(s0a, s0b, agg1, asn, batch3d, wc, bc, wo, bo):
    return pl.pallas_call(
        _tc2_body,
        grid=(_NB,),
        in_specs=[
            _row_spec(_R, D), _row_spec(_R, D), _row_spec(_R, D),
            _row_spec(_R, 2),
            pl.BlockSpec((1, 1, _R), lambda i: (i, 0, 0)),
            _full_spec((D, H)), _full_spec((1, H)),
            _full_spec((D, H)), _full_spec((1, H)),
        ],
        out_specs=[_full_spec((G, H)), _full_spec((G, H))],
        out_shape=[
            jax.ShapeDtypeStruct((G, H), _f32),
            jax.ShapeDtypeStruct((G, H), _f32),
        ],
    )(s0a, s0b, agg1, asn, batch3d, wc, bc, wo, bo)


def _tc3_call(g1, gm, gres, pmat, *weights):
    two = 2
    h_co, h_m, h_shuf, h1, h_res = pl.pallas_call(
        _tc3_body,
        out_shape=[
            jax.ShapeDtypeStruct((G, two), _f32),
            jax.ShapeDtypeStruct((G, two), _f32),
            jax.ShapeDtypeStruct((G, two), _f32),
            jax.ShapeDtypeStruct((G, two), _f32),
            jax.ShapeDtypeStruct((G, two), _f32),
        ],
    )(g1, gm, gres, pmat, *weights)
    return h1, h_m, h_co, h_res, h_shuf


def _tc4_call(g1, gm, w3, b3):
    return pl.pallas_call(
        _tc4_body,
        out_shape=[
            jax.ShapeDtypeStruct((G, H), _f32),
            jax.ShapeDtypeStruct((G, H), _f32),
        ],
    )(g1, gm, w3, b3)


def kernel(x, edge_index, batch, y, perm,
           W_gcn, b_gcn, W_ctx, b_ctx, W_obj, b_obj,
           W_ib, b_ib, W1, b1, W4, b4, W3, b3,
           Wc1, bc1, Wc2, bc2, Wo1, bo1, Wo2, bo2, Ws1, bs1, Ws2, bs2):
    src = edge_index[0]
    dst = edge_index[1]
    zeros_nd = jnp.zeros((N, D), _f32)
    batch3d = batch.reshape(_NB, 1, _R)
    r2 = lambda b: b.reshape(1, -1)

    segsum = _make_segsum1()
    agg_parts = segsum(x, src, dst, zeros_nd)
    asn, xs0, agg1, g1 = _tc1_call(
        agg_parts[0], agg_parts[1], x, batch3d,
        W_gcn, r2(b_gcn), W_ib, r2(b_ib))

    s_parts = segsum(xs0, src, dst, zeros_nd)
    g_m, g_res = _tc2_call(
        s_parts[0], s_parts[1], agg1, asn, batch3d,
        W_ctx, r2(b_ctx), W_obj, r2(b_obj))

    diff = perm.astype(_f32)[:, None] - jnp.arange(G, dtype=_f32)[None, :]
    pmat = jnp.maximum(1.0 - jnp.abs(diff), 0.0)
    h1, h_m, h_co, h_res, h_shuf = _tc3_call(
        g1, g_m, g_res, pmat,
        W1, r2(b1), W4, r2(b4),
        Wc1, r2(bc1), Wc2, r2(bc2), Wo1, r2(bo1), Wo2, r2(bo2),
        Ws1, r2(bs1), Ws2, r2(bs2))
    proj1, proj_m = _tc4_call(g1, g_m, W3, r2(b3))

    y_shuf = jnp.take(y, perm, axis=0)
    return (h1, g1, h_m, g_m, proj1, proj_m, asn, h_co, h_res, h_shuf, y_shuf)
